# Initial kernel scaffold; baseline (speedup 1.0000x reference)
import jax
import jax.numpy as jnp
from jax.experimental import pallas as pl


def kernel(images):
    raise NotImplementedError("write your pallas kernel here")


# speedup vs baseline: 3.4196x; 3.4196x over previous
import functools

import jax
import jax.numpy as jnp
from jax.experimental import pallas as pl
from jax.experimental.pallas import tpu as pltpu

NEG = -3e38


def _shift_right(x, d, fill):
    pad = jnp.full(x.shape[:-1] + (d,), fill, x.dtype)
    return jnp.concatenate([pad, x[..., :-d]], axis=-1)


def _cumsum_last(x):
    for d in (1, 2, 4, 8, 16, 32, 64):
        x = x + _shift_right(x, d, 0.0)
    return x


def _cummax_last(x):
    for d in (1, 2, 4, 8, 16, 32, 64):
        x = jnp.maximum(x, _shift_right(x, d, NEG))
    return x


def _dp_kernel(img_ref, out_ref):
    Bb, I, J = img_ref.shape

    col0_mask = jax.lax.broadcasted_iota(jnp.int32, (Bb, J), 1) == 0

    def thr_and_S(b):
        t = 0.5 * (_shift_right(b, 1, 0.0) + b)
        t = jnp.where(col0_mask, 0.0, t)
        return _cumsum_last(t)

    r0 = img_ref[:, 0, :]
    S0 = thr_and_S(r0)
    start = r0[:, 0:1]
    row = S0 + start

    def body(i, carry):
        row, a = carry
        b = img_ref[:, i, :]
        half_b = 0.5 * b
        cand_up = row + 0.5 * a + half_b
        cand_diag = _shift_right(row + 0.5 * a, 1, NEG) + half_b
        base = jnp.maximum(cand_up, cand_diag)
        S = thr_and_S(b)
        row = S + _cummax_last(base - S)
        return row, b

    row, _ = jax.lax.fori_loop(1, I, body, (row, r0))
    out_ref[:, 0] = row[:, J - 1]


@jax.jit
def kernel(images):
    B, I, J = images.shape
    Bb = 128
    grid = (B // Bb,)
    out = pl.pallas_call(
        _dp_kernel,
        grid=grid,
        in_specs=[pl.BlockSpec((Bb, I, J), lambda b: (b, 0, 0))],
        out_specs=pl.BlockSpec((Bb, 1), lambda b: (b, 0)),
        out_shape=jax.ShapeDtypeStruct((B, 1), jnp.float32),
    )(images)
    return out[:, 0]


# speedup vs baseline: 6.1292x; 1.7924x over previous
import jax
import jax.numpy as jnp
from jax.experimental import pallas as pl
from jax.experimental.pallas import tpu as pltpu

NEG = -3e38
ROWS = 8


def _shift_right(x, d, fill):
    pad = jnp.full(x.shape[:-1] + (d,), fill, x.dtype)
    return jnp.concatenate([pad, x[..., :-d]], axis=-1)


def _cumsum_last(x):
    for d in (1, 2, 4, 8, 16, 32, 64):
        x = x + _shift_right(x, d, 0.0)
    return x


def _cummax_last(x):
    for d in (1, 2, 4, 8, 16, 32, 64):
        x = jnp.maximum(x, _shift_right(x, d, NEG))
    return x


def _dp_kernel(img_ref, out_ref, row_ref, prev_ref):
    Bb, R, J = img_ref.shape
    t = pl.program_id(1)

    col0_mask = jax.lax.broadcasted_iota(jnp.int32, (Bb, J), 1) == 0

    def thr_and_S(b):
        th = 0.5 * (_shift_right(b, 1, 0.0) + b)
        th = jnp.where(col0_mask, 0.0, th)
        return _cumsum_last(th)

    def row_update(row, half_a, b):
        half_b = 0.5 * b
        tmp = row + half_a
        cand_up = tmp + half_b
        cand_diag = _shift_right(tmp, 1, NEG) + half_b
        base = jnp.maximum(cand_up, cand_diag)
        S = thr_and_S(b)
        return S + _cummax_last(base - S), half_b

    @pl.when(t == 0)
    def _init():
        r0 = img_ref[:, 0, :]
        row = thr_and_S(r0) + r0[:, 0:1]
        half_a = 0.5 * r0
        for r in range(1, R):
            row, half_a = row_update(row, half_a, img_ref[:, r, :])
        row_ref[:, :] = row
        prev_ref[:, :] = 2.0 * half_a

    @pl.when(t != 0)
    def _step():
        row = row_ref[:, :]
        half_a = 0.5 * prev_ref[:, :]
        for r in range(R):
            row, half_a = row_update(row, half_a, img_ref[:, r, :])
        row_ref[:, :] = row
        prev_ref[:, :] = 2.0 * half_a

    out_ref[:, 0] = row_ref[:, J - 1]


@jax.jit
def kernel(images):
    B, I, J = images.shape
    Bb = 128
    grid = (B // Bb, I // ROWS)
    out = pl.pallas_call(
        _dp_kernel,
        grid=grid,
        in_specs=[pl.BlockSpec((Bb, ROWS, J), lambda b, t: (b, t, 0))],
        out_specs=pl.BlockSpec((Bb, 1), lambda b, t: (b, 0)),
        out_shape=jax.ShapeDtypeStruct((B, 1), jnp.float32),
        scratch_shapes=[
            pltpu.VMEM((Bb, J), jnp.float32),
            pltpu.VMEM((Bb, J), jnp.float32),
        ],
        compiler_params=pltpu.CompilerParams(
            dimension_semantics=("arbitrary", "arbitrary"),
        ),
    )(images)
    return out[:, 0]
